# baseline (speedup 1.0000x reference)
import functools

import jax
import jax.numpy as jnp
from jax import lax
from jax.experimental import pallas as pl
from jax.experimental.pallas import tpu as pltpu
from jax.experimental.pallas import tpu_sc as plsc

NC = 2
NS = 16
L = 16
NW = NC * NS
NUM_RESULTS = 1917
NUM_CLASSES = 91
A_PAD = 1920
NJ = A_PAD // L
MAX_BOXES = 10
NEG = -1e30
BIG = 2 ** 30

_mesh = plsc.VectorSubcoreMesh(
    core_axis_name="c", subcore_axis_name="s", num_cores=NC, num_subcores=NS
)


@functools.partial(
    pl.kernel,
    out_type=jax.ShapeDtypeStruct((90, 96), jnp.float32),
    mesh=_mesh,
    compiler_params=pltpu.CompilerParams(needs_layout_passes=False),
    scratch_types=[
        pltpu.VMEM((4, A_PAD), jnp.float32),
        pltpu.VMEM((4, A_PAD), jnp.float32),
        pltpu.VMEM((4, A_PAD), jnp.float32),
        pltpu.VMEM((A_PAD,), jnp.float32),
        pltpu.VMEM((3, A_PAD), jnp.float32),
        pltpu.VMEM((3, 96), jnp.float32),
        pltpu.SemaphoreType.DMA,
        pltpu.SemaphoreType.DMA,
    ],
)
def _nms_sc(loc_hbm, pri_hbm, log_hbm, out_hbm, loc_v, pri_v, by_v, ar_v, s_v,
            row_v, sem_lp, sem_lg):
    wid = lax.axis_index("s") * NC + lax.axis_index("c")
    lanes = lax.iota(jnp.int32, L)

    cs = [wid + 1 + NW * t for t in range(3)]
    ceff = [jnp.minimum(c, 90) for c in cs]

    h_lg = [
        pltpu.async_copy(
            log_hbm.at[pl.ds(ceff[t], 1)], s_v.at[pl.ds(t, 1)], sem_lg
        )
        for t in range(3)
    ]
    h_loc = pltpu.async_copy(loc_hbm, loc_v, sem_lp)
    h_pri = pltpu.async_copy(pri_hbm, pri_v, sem_lp)
    h_loc.wait()
    h_pri.wait()

    def decode(j):
        sl = pl.ds(j * L, L)
        l0 = loc_v[0, sl]
        l1 = loc_v[1, sl]
        l2 = loc_v[2, sl]
        l3 = loc_v[3, sl]
        p0 = pri_v[0, sl]
        p1 = pri_v[1, sl]
        p2 = pri_v[2, sl]
        p3 = pri_v[3, sl]
        yc = l0 / 10.0 * p2 + p0
        xc = l1 / 10.0 * p3 + p1
        h = jnp.exp(l2 / 5.0) * p2
        w = jnp.exp(l3 / 5.0) * p3
        ymin = yc - h / 2.0
        xmin = xc - w / 2.0
        ymax = yc + h / 2.0
        xmax = xc + w / 2.0
        by_v[0, sl] = ymin
        by_v[1, sl] = xmin
        by_v[2, sl] = ymax
        by_v[3, sl] = xmax
        ar_v[sl] = (ymax - ymin) * (xmax - xmin)

    plsc.parallel_loop(0, NJ, 1, unroll=2)(decode)

    for h in h_lg:
        h.wait()

    def sigmoid_thresh(j, mbs):
        sl = pl.ds(j * L, L)
        out = []
        for t in range(3):
            m, bi = mbs[2 * t], mbs[2 * t + 1]
            v = s_v[t, sl]
            sg = 1.0 / (1.0 + jnp.exp(-v))
            s = jnp.where(sg > 0.01, sg, NEG)
            s_v[t, sl] = s
            upd = s > m
            out += [jnp.where(upd, s, m), jnp.where(upd, j, bi)]
        return tuple(out)

    mneg = jnp.full((L,), NEG, jnp.float32)
    izero = jnp.zeros((L,), jnp.int32)
    mbs = plsc.parallel_loop(
        0, NJ, 1, unroll=2, carry=(mneg, izero, mneg, izero, mneg, izero)
    )(sigmoid_thresh)

    def nms_step(k, mbs):
        sels = []
        for t in range(3):
            m, bi = mbs[2 * t], mbs[2 * t + 1]
            M = jnp.max(m, axis=0)
            eidx = bi * L + lanes
            I = jnp.min(jnp.where(m == M, eidx, BIG), axis=0)
            iv = jnp.full((L,), I, jnp.int32)
            sel0 = plsc.load_gather(by_v, [izero, iv])
            sel1 = plsc.load_gather(by_v, [jnp.full((L,), 1, jnp.int32), iv])
            sel2 = plsc.load_gather(by_v, [jnp.full((L,), 2, jnp.int32), iv])
            sel3 = plsc.load_gather(by_v, [jnp.full((L,), 3, jnp.int32), iv])
            a1 = plsc.load_gather(ar_v, [iv])
            plsc.store_scatter(s_v, [jnp.full((L,), t, jnp.int32), iv], mneg)
            sels.append((M, I, sel0, sel1, sel2, sel3, a1))
            km = lanes == k
            row_v[t, pl.ds(0, L)] = jnp.where(
                km, jnp.full((L,), M, jnp.float32), row_v[t, pl.ds(0, L)])
            row_v[t, pl.ds(16, L)] = jnp.where(
                km, jnp.full((L,), I.astype(jnp.float32), jnp.float32),
                row_v[t, pl.ds(16, L)])
            row_v[t, pl.ds(32, L)] = jnp.where(km, sel0, row_v[t, pl.ds(32, L)])
            row_v[t, pl.ds(48, L)] = jnp.where(km, sel1, row_v[t, pl.ds(48, L)])
            row_v[t, pl.ds(64, L)] = jnp.where(km, sel2, row_v[t, pl.ds(64, L)])
            row_v[t, pl.ds(80, L)] = jnp.where(km, sel3, row_v[t, pl.ds(80, L)])

        def suppress(j, mbs2):
            sl = pl.ds(j * L, L)
            b0 = by_v[0, sl]
            b1 = by_v[1, sl]
            b2 = by_v[2, sl]
            b3 = by_v[3, sl]
            ar = ar_v[sl]
            out = []
            for t in range(3):
                m2, bi2 = mbs2[2 * t], mbs2[2 * t + 1]
                _, _, sel0, sel1, sel2, sel3, a1 = sels[t]
                iy = jnp.maximum(sel0, b0)
                ix = jnp.maximum(sel1, b1)
                ay = jnp.minimum(sel2, b2)
                ax = jnp.minimum(sel3, b3)
                inter = jnp.maximum(ay - iy, 0.0) * jnp.maximum(ax - ix, 0.0)
                iou = inter / (a1 + ar - inter + 1e-8)
                s = jnp.where(iou >= 0.5, NEG, s_v[t, sl])
                s_v[t, sl] = s
                upd = s > m2
                out += [jnp.where(upd, s, m2), jnp.where(upd, j, bi2)]
            return tuple(out)

        return plsc.parallel_loop(
            0, NJ, 1, unroll=3, carry=(mneg, izero, mneg, izero, mneg, izero)
        )(suppress)

    lax.fori_loop(0, MAX_BOXES, nms_step, mbs)
    hs = [
        pltpu.async_copy(
            row_v.at[pl.ds(t, 1)], out_hbm.at[pl.ds(ceff[t] - 1, 1)], sem_lp
        )
        for t in range(3)
    ]
    for h in hs:
        h.wait()


def _merge_body(x_ref, o_ref):
    x = x_ref[...]
    sc = x[:, 0:10]
    idxm = x[:, 16:26]
    b0 = x[:, 32:42]
    b1 = x[:, 48:58]
    b2 = x[:, 64:74]
    b3 = x[:, 80:90]
    sc = jnp.where(sc > 0.0, sc, -1.0)
    rows = lax.broadcasted_iota(jnp.int32, (90, 10), 0)
    cols = lax.broadcasted_iota(jnp.int32, (90, 10), 1)
    flat = rows * 10 + cols
    clsm = (rows + 1).astype(jnp.float32)
    r16 = lax.broadcasted_iota(jnp.int32, (16, 128), 0)
    c128 = lax.broadcasted_iota(jnp.int32, (16, 128), 1)

    def step(k, carry):
        s, acc = carry
        M = jnp.max(s)
        f = jnp.min(jnp.where(s == M, flat, BIG))
        oh = flat == f
        vals = (
            jnp.sum(jnp.where(oh, s, 0.0)),
            jnp.sum(jnp.where(oh, idxm, 0.0)),
            jnp.sum(jnp.where(oh, clsm, 0.0)),
            jnp.sum(jnp.where(oh, b0, 0.0)),
            jnp.sum(jnp.where(oh, b1, 0.0)),
            jnp.sum(jnp.where(oh, b2, 0.0)),
            jnp.sum(jnp.where(oh, b3, 0.0)),
        )
        for j in range(7):
            acc = jnp.where((r16 == k) & (c128 == j), vals[j], acc)
        s = jnp.where(oh, -2.0, s)
        return s, acc

    _, acc = lax.fori_loop(0, MAX_BOXES, step, (sc, jnp.zeros((16, 128), jnp.float32)))
    o_ref[...] = acc[0:10, 0:7]


_merge = pl.pallas_call(
    _merge_body,
    out_shape=jax.ShapeDtypeStruct((10, 7), jnp.float32),
)


def kernel(inputs, locations, class_logits, priors):
    del inputs
    locT = jnp.zeros((4, A_PAD), jnp.float32).at[:, :NUM_RESULTS].set(locations.T)
    priT = jnp.zeros((4, A_PAD), jnp.float32).at[:, :NUM_RESULTS].set(priors)
    logT = jnp.full((NUM_CLASSES, A_PAD), -100.0, jnp.float32).at[:, :NUM_RESULTS].set(
        class_logits.T
    )
    cand = _nms_sc(locT, priT, logT)
    return _merge(cand)


# speedup vs baseline: 1.0260x; 1.0260x over previous
import functools

import jax
import jax.numpy as jnp
from jax import lax
from jax.experimental import pallas as pl
from jax.experimental.pallas import tpu as pltpu
from jax.experimental.pallas import tpu_sc as plsc

NC = 2
NS = 16
L = 16
NW = NC * NS
NUM_RESULTS = 1917
NUM_CLASSES = 91
A_PAD = 1920
NJ = A_PAD // L
MAX_BOXES = 10
NEG = -1e30
BIG = 2 ** 30

_mesh = plsc.VectorSubcoreMesh(
    core_axis_name="c", subcore_axis_name="s", num_cores=NC, num_subcores=NS
)


@functools.partial(
    pl.kernel,
    out_type=jax.ShapeDtypeStruct((90, 96), jnp.float32),
    mesh=_mesh,
    compiler_params=pltpu.CompilerParams(needs_layout_passes=False),
    scratch_types=[
        pltpu.VMEM((4, A_PAD), jnp.float32),
        pltpu.VMEM((4, A_PAD), jnp.float32),
        pltpu.VMEM((4, A_PAD), jnp.float32),
        pltpu.VMEM((A_PAD,), jnp.float32),
        pltpu.VMEM((3, A_PAD), jnp.float32),
        pltpu.VMEM((3, 96), jnp.float32),
        pltpu.SemaphoreType.DMA,
        pltpu.SemaphoreType.DMA,
    ],
)
def _nms_sc(loc_hbm, pri_hbm, log_hbm, out_hbm, loc_v, pri_v, by_v, ar_v, s_v,
            row_v, sem_lp, sem_lg):
    wid = lax.axis_index("s") * NC + lax.axis_index("c")
    lanes = lax.iota(jnp.int32, L)

    cs = [wid + 1 + NW * t for t in range(3)]
    ceff = [jnp.minimum(c, 90) for c in cs]

    h_lg = [
        pltpu.async_copy(
            log_hbm.at[pl.ds(ceff[t], 1)], s_v.at[pl.ds(t, 1)], sem_lg
        )
        for t in range(3)
    ]
    h_loc = pltpu.async_copy(loc_hbm, loc_v, sem_lp)
    h_pri = pltpu.async_copy(pri_hbm, pri_v, sem_lp)
    h_loc.wait()
    h_pri.wait()

    def decode(j):
        sl = pl.ds(j * L, L)
        l0 = loc_v[0, sl]
        l1 = loc_v[1, sl]
        l2 = loc_v[2, sl]
        l3 = loc_v[3, sl]
        p0 = pri_v[0, sl]
        p1 = pri_v[1, sl]
        p2 = pri_v[2, sl]
        p3 = pri_v[3, sl]
        yc = l0 / 10.0 * p2 + p0
        xc = l1 / 10.0 * p3 + p1
        h = jnp.exp(l2 / 5.0) * p2
        w = jnp.exp(l3 / 5.0) * p3
        ymin = yc - h / 2.0
        xmin = xc - w / 2.0
        ymax = yc + h / 2.0
        xmax = xc + w / 2.0
        by_v[0, sl] = ymin
        by_v[1, sl] = xmin
        by_v[2, sl] = ymax
        by_v[3, sl] = xmax
        ar_v[sl] = (ymax - ymin) * (xmax - xmin)

    plsc.parallel_loop(0, NJ, 1, unroll=2)(decode)

    for h in h_lg:
        h.wait()

    def sigmoid_thresh(j, mbs):
        sl = pl.ds(j * L, L)
        out = []
        for t in range(3):
            m, bi = mbs[2 * t], mbs[2 * t + 1]
            v = s_v[t, sl]
            sg = 1.0 / (1.0 + jnp.exp(-v))
            s = jnp.where(sg > 0.01, sg, NEG)
            s_v[t, sl] = s
            upd = s > m
            out += [jnp.where(upd, s, m), jnp.where(upd, j, bi)]
        return tuple(out)

    mneg = jnp.full((L,), NEG, jnp.float32)
    izero = jnp.zeros((L,), jnp.int32)
    mbs = plsc.parallel_loop(
        0, NJ, 1, unroll=2, carry=(mneg, izero, mneg, izero, mneg, izero)
    )(sigmoid_thresh)

    def nms_step(k, mbs):
        sels = []
        for t in range(3):
            m, bi = mbs[2 * t], mbs[2 * t + 1]
            M = jnp.max(m, axis=0)
            eidx = bi * L + lanes
            I = jnp.min(jnp.where(m == M, eidx, BIG), axis=0)
            iv = jnp.full((L,), I, jnp.int32)
            sel0 = plsc.load_gather(by_v, [izero, iv])
            sel1 = plsc.load_gather(by_v, [jnp.full((L,), 1, jnp.int32), iv])
            sel2 = plsc.load_gather(by_v, [jnp.full((L,), 2, jnp.int32), iv])
            sel3 = plsc.load_gather(by_v, [jnp.full((L,), 3, jnp.int32), iv])
            a1 = plsc.load_gather(ar_v, [iv])
            plsc.store_scatter(s_v, [jnp.full((L,), t, jnp.int32), iv], mneg)
            sels.append((M, I, sel0, sel1, sel2, sel3, a1))
            km = lanes == k
            row_v[t, pl.ds(0, L)] = jnp.where(
                km, jnp.full((L,), M, jnp.float32), row_v[t, pl.ds(0, L)])
            row_v[t, pl.ds(16, L)] = jnp.where(
                km, jnp.full((L,), I.astype(jnp.float32), jnp.float32),
                row_v[t, pl.ds(16, L)])
            row_v[t, pl.ds(32, L)] = jnp.where(km, sel0, row_v[t, pl.ds(32, L)])
            row_v[t, pl.ds(48, L)] = jnp.where(km, sel1, row_v[t, pl.ds(48, L)])
            row_v[t, pl.ds(64, L)] = jnp.where(km, sel2, row_v[t, pl.ds(64, L)])
            row_v[t, pl.ds(80, L)] = jnp.where(km, sel3, row_v[t, pl.ds(80, L)])

        def suppress(j, mbs2):
            sl = pl.ds(j * L, L)
            b0 = by_v[0, sl]
            b1 = by_v[1, sl]
            b2 = by_v[2, sl]
            b3 = by_v[3, sl]
            ar = ar_v[sl]
            out = []
            for t in range(3):
                m2, bi2 = mbs2[2 * t], mbs2[2 * t + 1]
                _, _, sel0, sel1, sel2, sel3, a1 = sels[t]
                iy = jnp.maximum(sel0, b0)
                ix = jnp.maximum(sel1, b1)
                ay = jnp.minimum(sel2, b2)
                ax = jnp.minimum(sel3, b3)
                inter = jnp.maximum(ay - iy, 0.0) * jnp.maximum(ax - ix, 0.0)
                iou = inter / (a1 + ar - inter + 1e-8)
                s = jnp.where(iou >= 0.5, NEG, s_v[t, sl])
                s_v[t, sl] = s
                upd = s > m2
                out += [jnp.where(upd, s, m2), jnp.where(upd, j, bi2)]
            return tuple(out)

        return plsc.parallel_loop(
            0, NJ, 1, unroll=2, carry=(mneg, izero, mneg, izero, mneg, izero)
        )(suppress)

    lax.fori_loop(0, MAX_BOXES, nms_step, mbs)
    hs = [
        pltpu.async_copy(
            row_v.at[pl.ds(t, 1)], out_hbm.at[pl.ds(ceff[t] - 1, 1)], sem_lp
        )
        for t in range(3)
    ]
    for h in hs:
        h.wait()


def _merge_body(x_ref, o_ref):
    x = x_ref[...]
    sc = x[:, 0:10]
    idxm = x[:, 16:26]
    b0 = x[:, 32:42]
    b1 = x[:, 48:58]
    b2 = x[:, 64:74]
    b3 = x[:, 80:90]
    sc = jnp.where(sc > 0.0, sc, -1.0)
    rows = lax.broadcasted_iota(jnp.int32, (90, 10), 0)
    cols = lax.broadcasted_iota(jnp.int32, (90, 10), 1)
    flat = rows * 10 + cols
    clsm = (rows + 1).astype(jnp.float32)
    r16 = lax.broadcasted_iota(jnp.int32, (16, 128), 0)
    c128 = lax.broadcasted_iota(jnp.int32, (16, 128), 1)

    def step(k, carry):
        s, acc = carry
        M = jnp.max(s)
        f = jnp.min(jnp.where(s == M, flat, BIG))
        oh = flat == f
        vals = (
            jnp.sum(jnp.where(oh, s, 0.0)),
            jnp.sum(jnp.where(oh, idxm, 0.0)),
            jnp.sum(jnp.where(oh, clsm, 0.0)),
            jnp.sum(jnp.where(oh, b0, 0.0)),
            jnp.sum(jnp.where(oh, b1, 0.0)),
            jnp.sum(jnp.where(oh, b2, 0.0)),
            jnp.sum(jnp.where(oh, b3, 0.0)),
        )
        for j in range(7):
            acc = jnp.where((r16 == k) & (c128 == j), vals[j], acc)
        s = jnp.where(oh, -2.0, s)
        return s, acc

    _, acc = lax.fori_loop(0, MAX_BOXES, step, (sc, jnp.zeros((16, 128), jnp.float32)))
    o_ref[...] = acc[0:10, 0:7]


_merge = pl.pallas_call(
    _merge_body,
    out_shape=jax.ShapeDtypeStruct((10, 7), jnp.float32),
)


def kernel(inputs, locations, class_logits, priors):
    del inputs
    locT = jnp.zeros((4, A_PAD), jnp.float32).at[:, :NUM_RESULTS].set(locations.T)
    priT = jnp.zeros((4, A_PAD), jnp.float32).at[:, :NUM_RESULTS].set(priors)
    logT = jnp.full((NUM_CLASSES, A_PAD), -100.0, jnp.float32).at[:, :NUM_RESULTS].set(
        class_logits.T
    )
    cand = _nms_sc(locT, priT, logT)
    return _merge(cand)


# speedup vs baseline: 1.0541x; 1.0274x over previous
import functools

import jax
import jax.numpy as jnp
from jax import lax
from jax.experimental import pallas as pl
from jax.experimental.pallas import tpu as pltpu
from jax.experimental.pallas import tpu_sc as plsc

NC = 2
NS = 16
L = 16
NW = NC * NS
NUM_RESULTS = 1917
NUM_CLASSES = 91
A_PAD = 1920
NJ = A_PAD // L
MAX_BOXES = 10
NEG = -1e30
BIG = 2 ** 30

_mesh = plsc.VectorSubcoreMesh(
    core_axis_name="c", subcore_axis_name="s", num_cores=NC, num_subcores=NS
)


@functools.partial(
    pl.kernel,
    out_type=jax.ShapeDtypeStruct((90, 96), jnp.float32),
    mesh=_mesh,
    compiler_params=pltpu.CompilerParams(needs_layout_passes=False),
    scratch_types=[
        pltpu.VMEM((4, A_PAD), jnp.float32),
        pltpu.VMEM((4, A_PAD), jnp.float32),
        pltpu.VMEM((4, A_PAD), jnp.float32),
        pltpu.VMEM((A_PAD,), jnp.float32),
        pltpu.VMEM((3, A_PAD), jnp.float32),
        pltpu.VMEM((3, 96), jnp.float32),
        pltpu.SemaphoreType.DMA,
        pltpu.SemaphoreType.DMA,
    ],
)
def _nms_sc(loc_hbm, pri_hbm, log_hbm, out_hbm, loc_v, pri_v, by_v, ar_v, s_v,
            row_v, sem_lp, sem_lg):
    wid = lax.axis_index("s") * NC + lax.axis_index("c")
    lanes = lax.iota(jnp.int32, L)

    cs = [wid + 1 + NW * t for t in range(3)]
    ceff = [jnp.minimum(c, 90) for c in cs]

    h_lg = [
        pltpu.async_copy(
            log_hbm.at[pl.ds(ceff[t], 1)], s_v.at[pl.ds(t, 1)], sem_lg
        )
        for t in range(3)
    ]
    h_loc = pltpu.async_copy(loc_hbm, loc_v, sem_lp)
    h_pri = pltpu.async_copy(pri_hbm, pri_v, sem_lp)
    h_loc.wait()
    h_pri.wait()

    def decode(j):
        sl = pl.ds(j * L, L)
        l0 = loc_v[0, sl]
        l1 = loc_v[1, sl]
        l2 = loc_v[2, sl]
        l3 = loc_v[3, sl]
        p0 = pri_v[0, sl]
        p1 = pri_v[1, sl]
        p2 = pri_v[2, sl]
        p3 = pri_v[3, sl]
        yc = l0 / 10.0 * p2 + p0
        xc = l1 / 10.0 * p3 + p1
        h = jnp.exp(l2 / 5.0) * p2
        w = jnp.exp(l3 / 5.0) * p3
        ymin = yc - h / 2.0
        xmin = xc - w / 2.0
        ymax = yc + h / 2.0
        xmax = xc + w / 2.0
        by_v[0, sl] = ymin
        by_v[1, sl] = xmin
        by_v[2, sl] = ymax
        by_v[3, sl] = xmax
        ar_v[sl] = (ymax - ymin) * (xmax - xmin)

    plsc.parallel_loop(0, NJ, 1, unroll=2)(decode)

    for h in h_lg:
        h.wait()

    def sigmoid_thresh(j, mbs):
        sl = pl.ds(j * L, L)
        out = []
        for t in range(3):
            m, bi = mbs[2 * t], mbs[2 * t + 1]
            v = s_v[t, sl]
            sg = 1.0 / (1.0 + jnp.exp(-v))
            s = jnp.where(sg > 0.01, sg, NEG)
            s_v[t, sl] = s
            upd = s > m
            out += [jnp.where(upd, s, m), jnp.where(upd, j, bi)]
        return tuple(out)

    mneg = jnp.full((L,), NEG, jnp.float32)
    izero = jnp.zeros((L,), jnp.int32)
    mbs = plsc.parallel_loop(
        0, NJ, 1, unroll=2, carry=(mneg, izero, mneg, izero, mneg, izero)
    )(sigmoid_thresh)

    def pick(k, mbs):
        sels = []
        for t in range(3):
            m, bi = mbs[2 * t], mbs[2 * t + 1]
            M = jnp.max(m, axis=0)
            eidx = bi * L + lanes
            I = jnp.min(jnp.where(m == M, eidx, BIG), axis=0)
            iv = jnp.full((L,), I, jnp.int32)
            sel0 = plsc.load_gather(by_v, [izero, iv])
            sel1 = plsc.load_gather(by_v, [jnp.full((L,), 1, jnp.int32), iv])
            sel2 = plsc.load_gather(by_v, [jnp.full((L,), 2, jnp.int32), iv])
            sel3 = plsc.load_gather(by_v, [jnp.full((L,), 3, jnp.int32), iv])
            a1 = plsc.load_gather(ar_v, [iv])
            plsc.store_scatter(s_v, [jnp.full((L,), t, jnp.int32), iv], mneg)
            sels.append((M, I, sel0, sel1, sel2, sel3, a1))
            km = lanes == k
            row_v[t, pl.ds(0, L)] = jnp.where(
                km, jnp.full((L,), M, jnp.float32), row_v[t, pl.ds(0, L)])
            row_v[t, pl.ds(16, L)] = jnp.where(
                km, jnp.full((L,), I.astype(jnp.float32), jnp.float32),
                row_v[t, pl.ds(16, L)])
            row_v[t, pl.ds(32, L)] = jnp.where(km, sel0, row_v[t, pl.ds(32, L)])
            row_v[t, pl.ds(48, L)] = jnp.where(km, sel1, row_v[t, pl.ds(48, L)])
            row_v[t, pl.ds(64, L)] = jnp.where(km, sel2, row_v[t, pl.ds(64, L)])
            row_v[t, pl.ds(80, L)] = jnp.where(km, sel3, row_v[t, pl.ds(80, L)])
        return sels

    def nms_step(k, mbs):
        sels = pick(k, mbs)

        def suppress(j, mbs2):
            sl = pl.ds(j * L, L)
            b0 = by_v[0, sl]
            b1 = by_v[1, sl]
            b2 = by_v[2, sl]
            b3 = by_v[3, sl]
            ar = ar_v[sl]
            out = []
            for t in range(3):
                m2, bi2 = mbs2[2 * t], mbs2[2 * t + 1]
                _, _, sel0, sel1, sel2, sel3, a1 = sels[t]
                iy = jnp.maximum(sel0, b0)
                ix = jnp.maximum(sel1, b1)
                ay = jnp.minimum(sel2, b2)
                ax = jnp.minimum(sel3, b3)
                inter = jnp.maximum(ay - iy, 0.0) * jnp.maximum(ax - ix, 0.0)
                iou = inter / (a1 + ar - inter + 1e-8)
                s = jnp.where(iou >= 0.5, NEG, s_v[t, sl])
                s_v[t, sl] = s
                upd = s > m2
                out += [jnp.where(upd, s, m2), jnp.where(upd, j, bi2)]
            return tuple(out)

        return plsc.parallel_loop(
            0, NJ, 1, unroll=2, carry=(mneg, izero, mneg, izero, mneg, izero)
        )(suppress)

    mbs = lax.fori_loop(0, MAX_BOXES - 1, nms_step, mbs)
    pick(MAX_BOXES - 1, mbs)
    hs = [
        pltpu.async_copy(
            row_v.at[pl.ds(t, 1)], out_hbm.at[pl.ds(ceff[t] - 1, 1)], sem_lp
        )
        for t in range(3)
    ]
    for h in hs:
        h.wait()


def _merge_body(x_ref, o_ref):
    x = x_ref[...]
    sc = x[:, 0:10]
    idxm = x[:, 16:26]
    b0 = x[:, 32:42]
    b1 = x[:, 48:58]
    b2 = x[:, 64:74]
    b3 = x[:, 80:90]
    sc = jnp.where(sc > 0.0, sc, -1.0)
    rows = lax.broadcasted_iota(jnp.int32, (90, 10), 0)
    cols = lax.broadcasted_iota(jnp.int32, (90, 10), 1)
    flat = rows * 10 + cols
    clsm = (rows + 1).astype(jnp.float32)
    r16 = lax.broadcasted_iota(jnp.int32, (16, 128), 0)
    c128 = lax.broadcasted_iota(jnp.int32, (16, 128), 1)

    def step(k, carry):
        s, acc = carry
        M = jnp.max(s)
        f = jnp.min(jnp.where(s == M, flat, BIG))
        oh = flat == f
        vals = (
            jnp.sum(jnp.where(oh, s, 0.0)),
            jnp.sum(jnp.where(oh, idxm, 0.0)),
            jnp.sum(jnp.where(oh, clsm, 0.0)),
            jnp.sum(jnp.where(oh, b0, 0.0)),
            jnp.sum(jnp.where(oh, b1, 0.0)),
            jnp.sum(jnp.where(oh, b2, 0.0)),
            jnp.sum(jnp.where(oh, b3, 0.0)),
        )
        for j in range(7):
            acc = jnp.where((r16 == k) & (c128 == j), vals[j], acc)
        s = jnp.where(oh, -2.0, s)
        return s, acc

    _, acc = lax.fori_loop(0, MAX_BOXES, step, (sc, jnp.zeros((16, 128), jnp.float32)))
    o_ref[...] = acc[0:10, 0:7]


_merge = pl.pallas_call(
    _merge_body,
    out_shape=jax.ShapeDtypeStruct((10, 7), jnp.float32),
)


def kernel(inputs, locations, class_logits, priors):
    del inputs
    locT = jnp.zeros((4, A_PAD), jnp.float32).at[:, :NUM_RESULTS].set(locations.T)
    priT = jnp.zeros((4, A_PAD), jnp.float32).at[:, :NUM_RESULTS].set(priors)
    logT = jnp.full((NUM_CLASSES, A_PAD), -100.0, jnp.float32).at[:, :NUM_RESULTS].set(
        class_logits.T
    )
    cand = _nms_sc(locT, priT, logT)
    return _merge(cand)


# speedup vs baseline: 1.0654x; 1.0107x over previous
import functools

import jax
import jax.numpy as jnp
from jax import lax
from jax.experimental import pallas as pl
from jax.experimental.pallas import tpu as pltpu
from jax.experimental.pallas import tpu_sc as plsc

NC = 2
NS = 16
L = 16
NW = NC * NS
NUM_RESULTS = 1917
NUM_CLASSES = 91
A_PAD = 1920
NJ = A_PAD // L
MAX_BOXES = 10
NEG = -1e30
BIG = 2 ** 30

_mesh = plsc.VectorSubcoreMesh(
    core_axis_name="c", subcore_axis_name="s", num_cores=NC, num_subcores=NS
)


@functools.partial(
    pl.kernel,
    out_type=jax.ShapeDtypeStruct((90, 96), jnp.float32),
    mesh=_mesh,
    compiler_params=pltpu.CompilerParams(needs_layout_passes=False),
    scratch_types=[
        pltpu.VMEM((4, A_PAD), jnp.float32),
        pltpu.VMEM((4, A_PAD), jnp.float32),
        pltpu.VMEM((4, A_PAD), jnp.float32),
        pltpu.VMEM((A_PAD,), jnp.float32),
        pltpu.VMEM((3, A_PAD), jnp.float32),
        pltpu.VMEM((3, 96), jnp.float32),
        pltpu.SemaphoreType.DMA,
        pltpu.SemaphoreType.DMA,
    ],
)
def _nms_sc(loc_hbm, pri_hbm, log_hbm, out_hbm, loc_v, pri_v, by_v, ar_v, s_v,
            row_v, sem_lp, sem_lg):
    wid = lax.axis_index("s") * NC + lax.axis_index("c")
    lanes = lax.iota(jnp.int32, L)

    cs = [wid + 1 + NW * t for t in range(3)]
    ceff = [jnp.minimum(c, 90) for c in cs]

    h_lg = [
        pltpu.async_copy(
            log_hbm.at[pl.ds(ceff[t], 1)], s_v.at[pl.ds(t, 1)], sem_lg
        )
        for t in range(3)
    ]
    h_loc = pltpu.async_copy(loc_hbm, loc_v, sem_lp)
    h_pri = pltpu.async_copy(pri_hbm, pri_v, sem_lp)
    h_loc.wait()
    h_pri.wait()

    def decode(j):
        sl = pl.ds(j * L, L)
        l0 = loc_v[0, sl]
        l1 = loc_v[1, sl]
        l2 = loc_v[2, sl]
        l3 = loc_v[3, sl]
        p0 = pri_v[0, sl]
        p1 = pri_v[1, sl]
        p2 = pri_v[2, sl]
        p3 = pri_v[3, sl]
        yc = l0 / 10.0 * p2 + p0
        xc = l1 / 10.0 * p3 + p1
        h = jnp.exp(l2 / 5.0) * p2
        w = jnp.exp(l3 / 5.0) * p3
        ymin = yc - h / 2.0
        xmin = xc - w / 2.0
        ymax = yc + h / 2.0
        xmax = xc + w / 2.0
        by_v[0, sl] = ymin
        by_v[1, sl] = xmin
        by_v[2, sl] = ymax
        by_v[3, sl] = xmax
        ar_v[sl] = (ymax - ymin) * (xmax - xmin)

    plsc.parallel_loop(0, NJ, 1, unroll=2)(decode)

    for h in h_lg:
        h.wait()

    def sigmoid_thresh(j, mbs):
        sl = pl.ds(j * L, L)
        out = []
        for t in range(3):
            m, bi = mbs[2 * t], mbs[2 * t + 1]
            v = s_v[t, sl]
            sg = 1.0 / (1.0 + jnp.exp(-v))
            s = jnp.where(sg > 0.01, sg, NEG)
            s_v[t, sl] = s
            upd = s > m
            out += [jnp.where(upd, s, m), jnp.where(upd, j, bi)]
        return tuple(out)

    mneg = jnp.full((L,), NEG, jnp.float32)
    izero = jnp.zeros((L,), jnp.int32)
    mbs = plsc.parallel_loop(
        0, NJ, 1, unroll=2, carry=(mneg, izero, mneg, izero, mneg, izero)
    )(sigmoid_thresh)

    def pick(k, mbs):
        sels = []
        for t in range(3):
            m, bi = mbs[2 * t], mbs[2 * t + 1]
            M = jnp.max(m, axis=0)
            eidx = bi * L + lanes
            I = jnp.min(jnp.where(m == M, eidx, BIG), axis=0)
            iv = jnp.full((L,), I, jnp.int32)
            sel0 = plsc.load_gather(by_v, [izero, iv])
            sel1 = plsc.load_gather(by_v, [jnp.full((L,), 1, jnp.int32), iv])
            sel2 = plsc.load_gather(by_v, [jnp.full((L,), 2, jnp.int32), iv])
            sel3 = plsc.load_gather(by_v, [jnp.full((L,), 3, jnp.int32), iv])
            a1 = plsc.load_gather(ar_v, [iv])
            plsc.store_scatter(s_v, [jnp.full((L,), t, jnp.int32), iv], mneg)
            sels.append((M, I, sel0, sel1, sel2, sel3, a1))
            km = lanes == k
            row_v[t, pl.ds(0, L)] = jnp.where(
                km, jnp.full((L,), M, jnp.float32), row_v[t, pl.ds(0, L)])
            row_v[t, pl.ds(16, L)] = jnp.where(
                km, jnp.full((L,), I.astype(jnp.float32), jnp.float32),
                row_v[t, pl.ds(16, L)])
            row_v[t, pl.ds(32, L)] = jnp.where(km, sel0, row_v[t, pl.ds(32, L)])
            row_v[t, pl.ds(48, L)] = jnp.where(km, sel1, row_v[t, pl.ds(48, L)])
            row_v[t, pl.ds(64, L)] = jnp.where(km, sel2, row_v[t, pl.ds(64, L)])
            row_v[t, pl.ds(80, L)] = jnp.where(km, sel3, row_v[t, pl.ds(80, L)])
        return sels

    def nms_step(k, mbs):
        sels = pick(k, mbs)

        def suppress(j, mbs2):
            sl = pl.ds(j * L, L)
            b0 = by_v[0, sl]
            b1 = by_v[1, sl]
            b2 = by_v[2, sl]
            b3 = by_v[3, sl]
            ar = ar_v[sl]
            out = []
            for t in range(3):
                m2, bi2 = mbs2[2 * t], mbs2[2 * t + 1]
                _, _, sel0, sel1, sel2, sel3, a1 = sels[t]
                iy = jnp.maximum(sel0, b0)
                ix = jnp.maximum(sel1, b1)
                ay = jnp.minimum(sel2, b2)
                ax = jnp.minimum(sel3, b3)
                inter = jnp.maximum(ay - iy, 0.0) * jnp.maximum(ax - ix, 0.0)
                s = jnp.where(inter + inter >= a1 + ar - inter + 1e-8, NEG, s_v[t, sl])
                s_v[t, sl] = s
                upd = s > m2
                out += [jnp.where(upd, s, m2), jnp.where(upd, j, bi2)]
            return tuple(out)

        return plsc.parallel_loop(
            0, NJ, 1, unroll=2, carry=(mneg, izero, mneg, izero, mneg, izero)
        )(suppress)

    mbs = lax.fori_loop(0, MAX_BOXES - 1, nms_step, mbs)
    pick(MAX_BOXES - 1, mbs)
    hs = [
        pltpu.async_copy(
            row_v.at[pl.ds(t, 1)], out_hbm.at[pl.ds(ceff[t] - 1, 1)], sem_lp
        )
        for t in range(3)
    ]
    for h in hs:
        h.wait()


def _merge_body(x_ref, o_ref):
    x = x_ref[...]
    sc = x[:, 0:10]
    idxm = x[:, 16:26]
    b0 = x[:, 32:42]
    b1 = x[:, 48:58]
    b2 = x[:, 64:74]
    b3 = x[:, 80:90]
    sc = jnp.where(sc > 0.0, sc, -1.0)
    rows = lax.broadcasted_iota(jnp.int32, (90, 10), 0)
    cols = lax.broadcasted_iota(jnp.int32, (90, 10), 1)
    flat = rows * 10 + cols
    clsm = (rows + 1).astype(jnp.float32)
    r16 = lax.broadcasted_iota(jnp.int32, (16, 128), 0)
    c128 = lax.broadcasted_iota(jnp.int32, (16, 128), 1)

    def step(k, carry):
        s, acc = carry
        M = jnp.max(s)
        f = jnp.min(jnp.where(s == M, flat, BIG))
        oh = flat == f
        vals = (
            jnp.sum(jnp.where(oh, s, 0.0)),
            jnp.sum(jnp.where(oh, idxm, 0.0)),
            jnp.sum(jnp.where(oh, clsm, 0.0)),
            jnp.sum(jnp.where(oh, b0, 0.0)),
            jnp.sum(jnp.where(oh, b1, 0.0)),
            jnp.sum(jnp.where(oh, b2, 0.0)),
            jnp.sum(jnp.where(oh, b3, 0.0)),
        )
        for j in range(7):
            acc = jnp.where((r16 == k) & (c128 == j), vals[j], acc)
        s = jnp.where(oh, -2.0, s)
        return s, acc

    _, acc = lax.fori_loop(0, MAX_BOXES, step, (sc, jnp.zeros((16, 128), jnp.float32)))
    o_ref[...] = acc[0:10, 0:7]


_merge = pl.pallas_call(
    _merge_body,
    out_shape=jax.ShapeDtypeStruct((10, 7), jnp.float32),
)


def kernel(inputs, locations, class_logits, priors):
    del inputs
    locT = jnp.zeros((4, A_PAD), jnp.float32).at[:, :NUM_RESULTS].set(locations.T)
    priT = jnp.zeros((4, A_PAD), jnp.float32).at[:, :NUM_RESULTS].set(priors)
    logT = jnp.full((NUM_CLASSES, A_PAD), -100.0, jnp.float32).at[:, :NUM_RESULTS].set(
        class_logits.T
    )
    cand = _nms_sc(locT, priT, logT)
    return _merge(cand)
